# baseline (speedup 1.0000x reference)
import jax
import jax.numpy as jnp
from jax import lax
from jax.experimental import pallas as pl
from jax.experimental.pallas import tpu as pltpu
from jax.experimental.pallas import tpu_sc as plsc

_D = 256
_SCALE = 0.01
_OFFSET = -1.28
_N = 2_000_000
_C = 2000
_NCHUNKS = _N // _C
_NW = 32
_NSB = 5
_SB = _C // _NSB
_VSB = _SB // 16
_ML = 4 * _SB
_MFULL = _ML // 128
_MREM = _ML - _MFULL * 128
_OCAP = _ML + 64
_SX = _D * _D
_OFF4 = [0, _D, _SX, _SX + _D]


def _body(xs_hbm, ys_hbm, zs_hbm, grid8_hbm, out_hbm,
          xsb, ysb, zsb, wbuf, mcol, midx, mstg, oidx, osid, ostg,
          vlb, vrb, outb, sem0, sem1, sem_in):
    cid = lax.axis_index("c")
    sid = lax.axis_index("s")
    w = sid * 2 + cid
    nfull = _NCHUNKS // _NW
    nch = jnp.where(w < _NCHUNKS % _NW, nfull + 1, nfull)
    lane = lax.iota(jnp.int32, 16)
    zeros = jnp.zeros((16,), jnp.int32)

    @pl.loop(0, _OCAP // 16)
    def _zinit(t):
        oidx[0, pl.ds(t * 16, 16)] = zeros
        oidx[1, pl.ds(t * 16, 16)] = zeros

    def bucket(q):
        e0 = jnp.clip((q - _OFFSET) * (1.0 / _SCALE), 1.0, float(_D - 1))
        e0 = e0.astype(jnp.int32)
        f0 = e0.astype(jnp.float32)
        p0 = f0 * _SCALE + _OFFSET
        pm1 = (f0 - 1.0) * _SCALE + _OFFSET
        ir = jnp.where(pm1 >= q, e0 - 1, jnp.where(p0 >= q, e0, e0 + 1))
        ir = jnp.clip(ir, 1, _D - 1)
        il = ir - 1
        fil = il.astype(jnp.float32)
        pleft = fil * _SCALE + _OFFSET
        pright = (fil + 1.0) * _SCALE + _OFFSET
        dl = jnp.maximum(q - pleft, 0.0)
        dr = jnp.maximum(pright - q, 0.0)
        bz = (dl == 0.0) & (dr == 0.0)
        dl = jnp.where(bz, 1.0, dl)
        dr = jnp.where(bz, 1.0, dr)
        rcp = 1.0 / (dl + dr)
        return il, dr * rcp, dl * rcp

    @pl.loop(0, nch)
    def _chunk(g):
        base = (w + g * _NW) * _C
        cin = [pltpu.async_copy(xs_hbm.at[pl.ds(base, _C)], xsb, sem_in),
               pltpu.async_copy(ys_hbm.at[pl.ds(base, _C)], ysb, sem_in),
               pltpu.async_copy(zs_hbm.at[pl.ds(base, _C)], zsb, sem_in)]
        for cp in cin:
            cp.wait()

        def phase_a(sb):
            bk = sb % 2

            @plsc.parallel_loop(0, _VSB, carry=jnp.int32(0))
            def _pa(j, cb):
                sl = pl.ds(sb * _SB + j * 16, 16)
                xq = xsb[sl]
                yq = ysb[sl]
                zq = zsb[sl]
                ilx, fxl, fxr = bucket(xq)
                ily, fyl, fyr = bucket(yq)
                ilz, fzl, fzr = bucket(zq)
                wbuf[0, sl] = fxl
                wbuf[1, sl] = fxr
                wbuf[2, sl] = fyl
                wbuf[3, sl] = fyr
                wbuf[4, sl] = fzl
                wbuf[5, sl] = fzr
                fbase = ilx * _SX + ily * _D + ilz
                mz = ilz & 7
                mcol[bk, pl.ds(j * 16, 16)] = mz
                bm = mz == 7
                pcb = plsc.all_reduce_population_count(bm)[0]
                for c4 in range(4):
                    m = fbase + _OFF4[c4]
                    midx[bk, pl.ds(c4 * _SB + j * 16, 16)] = m >> 3
                    osl = pl.ds(cb + c4 * pcb, 16)
                    plsc.store_compressed(
                        oidx.at[bk, osl], (m + 1) >> 3, mask=bm)
                    plsc.store_compressed(
                        osid.at[bk, osl],
                        c4 * _C + sb * _SB + j * 16 + lane, mask=bm)
                return cb + 4 * pcb

            return _pa

        def fire(sb, cnt, sem):
            bk = sb % 2
            cps = [pltpu.async_copy(
                grid8_hbm.at[midx.at[bk, pl.ds((r % _MFULL) * 128, 128)]],
                mstg.at[bk, pl.ds((r % _MFULL) * 128, 128)], sem)
                for r in range(2 * _MFULL)]
            cps.append(pltpu.async_copy(
                grid8_hbm.at[midx.at[bk, pl.ds(_MFULL * 128, _MREM)]],
                mstg.at[bk, pl.ds(_MFULL * 128, _MREM)], sem))
            n_o = (cnt + 127) >> 7

            @pl.loop(0, n_o)
            def _fo(k):
                pltpu.async_copy(
                    grid8_hbm.at[oidx.at[bk, pl.ds(k * 128, 128)]],
                    ostg.at[bk, pl.ds(k * 128, 128)], sem)

            return cps, n_o

        def wait(sb, cps, n_o, sem):
            bk = sb % 2
            for cp in cps:
                cp.wait()

            @pl.loop(0, n_o)
            def _wo(k):
                pltpu.make_async_copy(
                    grid8_hbm.at[pl.ds(0, 128)],
                    ostg.at[bk, pl.ds(0, 128)], sem).wait()

        def scatter_back(sb, cnt):
            bk = sb % 2
            for c4 in range(4):
                @plsc.parallel_loop(0, _VSB)
                def _sm(t):
                    colv = mcol[bk, pl.ds(t * 16, 16)]
                    pos = c4 * _SB + t * 16 + lane
                    vl = plsc.load_gather(mstg.at[bk], [pos, colv])
                    vr = plsc.load_gather(
                        mstg.at[bk], [pos, jnp.minimum(colv + 1, 7)])
                    dsl = pl.ds(c4 * _C + sb * _SB + t * 16, 16)
                    vlb[dsl] = vl
                    vrb[dsl] = vr
            n_t = (cnt + 15) >> 4

            zc = jnp.zeros((16,), jnp.int32)

            @pl.loop(0, n_t)
            def _so(t):
                posn = t * 16 + lane
                mk = posn < cnt
                sidv = osid[bk, pl.ds(t * 16, 16)]
                v = plsc.load_gather(ostg.at[bk], [posn, zc], mask=mk)
                plsc.store_scatter(vrb, [sidv], v, mask=mk)

        def phase_b(sb):
            @plsc.parallel_loop(0, _VSB)
            def _pb(j):
                sl = pl.ds(sb * _SB + j * 16, 16)
                fxl = wbuf[0, sl]
                fxr = wbuf[1, sl]
                fyl = wbuf[2, sl]
                fyr = wbuf[3, sl]
                fzl = wbuf[4, sl]
                fzr = wbuf[5, sl]
                o = sb * _SB + j * 16
                a = []
                for c4 in range(4):
                    vl = vlb[pl.ds(c4 * _C + o, 16)]
                    vr = vrb[pl.ds(c4 * _C + o, 16)]
                    a.append(vl * fzl + vr * fzr)
                b0 = a[0] * fyl + a[1] * fyr
                b1 = a[2] * fyl + a[3] * fyr
                outb[pl.ds(o, 16)] = b0 * fxl + b1 * fxr

        sems = [sem0, sem1]
        inflight = {}
        cnt0 = phase_a(0)
        inflight[0] = (cnt0,) + fire(0, cnt0, sems[0])
        cnt1 = phase_a(1)
        inflight[1] = (cnt1,) + fire(1, cnt1, sems[1])
        for sb in range(2, _NSB):
            pcnt, pcps, pno = inflight.pop(sb - 2)
            wait(sb - 2, pcps, pno, sems[sb % 2])
            scatter_back(sb - 2, pcnt)
            phase_b(sb - 2)
            cnt = phase_a(sb)
            inflight[sb] = (cnt,) + fire(sb, cnt, sems[sb % 2])
        for sb in (_NSB - 2, _NSB - 1):
            pcnt, pcps, pno = inflight.pop(sb)
            wait(sb, pcps, pno, sems[sb % 2])
            scatter_back(sb, pcnt)
            phase_b(sb)

        pltpu.sync_copy(outb, out_hbm.at[pl.ds(base, _C)])


def kernel(x, sdf_grid, x_pts, y_pts, z_pts):
    x = x.reshape(-1, 3).astype(jnp.float32)
    n = x.shape[0]
    xs, ys, zs = x[:, 0], x[:, 1], x[:, 2]
    grid8 = sdf_grid.astype(jnp.float32).reshape(-1, 8)
    mesh = plsc.VectorSubcoreMesh(core_axis_name="c", subcore_axis_name="s")
    run = pl.kernel(
        _body,
        out_type=jax.ShapeDtypeStruct((n,), jnp.float32),
        mesh=mesh,
        compiler_params=pltpu.CompilerParams(
            needs_layout_passes=False, use_tc_tiling_on_sc=False),
        scratch_types=[
            pltpu.VMEM((_C,), jnp.float32),
            pltpu.VMEM((_C,), jnp.float32),
            pltpu.VMEM((_C,), jnp.float32),
            pltpu.VMEM((6, _C), jnp.float32),
            pltpu.VMEM((2, _SB), jnp.int32),
            pltpu.VMEM((2, _ML), jnp.int32),
            pltpu.VMEM((2, _ML, 8), jnp.float32),
            pltpu.VMEM((2, _OCAP), jnp.int32),
            pltpu.VMEM((2, _OCAP), jnp.int32),
            pltpu.VMEM((2, _OCAP, 8), jnp.float32),
            pltpu.VMEM((4 * _C,), jnp.float32),
            pltpu.VMEM((4 * _C,), jnp.float32),
            pltpu.VMEM((_C,), jnp.float32),
            pltpu.SemaphoreType.DMA,
            pltpu.SemaphoreType.DMA,
            pltpu.SemaphoreType.DMA,
        ],
    )
    return run(xs, ys, zs, grid8)


# speedup vs baseline: 1.3341x; 1.3341x over previous
import jax
import jax.numpy as jnp
from jax import lax
from jax.experimental import pallas as pl
from jax.experimental.pallas import tpu as pltpu
from jax.experimental.pallas import tpu_sc as plsc

_D = 256
_SCALE = 0.01
_OFFSET = -1.28
_N = 2_000_000
_C = 2000
_NCHUNKS = _N // _C
_NW = 32
_NSB = 5
_SB = _C // _NSB
_VSB = _SB // 16
_ML = 4 * _SB
_MFULL = _ML // 128
_MREM = _ML - _MFULL * 128
_OCAP = _ML + 64
_SX = _D * _D
_OFF4 = [0, _D, _SX, _SX + _D]


def _body(xs_hbm, ys_hbm, zs_hbm, grid8_hbm, out_hbm,
          xsb, ysb, zsb, wbuf, mcol, midx, mstg, oidx, osid, ostg,
          vrb, outb, sem0, sem1, sem_in):
    cid = lax.axis_index("c")
    sid = lax.axis_index("s")
    w = sid * 2 + cid
    nfull = _NCHUNKS // _NW
    nch = jnp.where(w < _NCHUNKS % _NW, nfull + 1, nfull)
    lane = lax.iota(jnp.int32, 16)
    zeros = jnp.zeros((16,), jnp.int32)

    @pl.loop(0, _OCAP // 16)
    def _zinit(t):
        oidx[0, pl.ds(t * 16, 16)] = zeros
        oidx[1, pl.ds(t * 16, 16)] = zeros

    def bucket(q):
        e0 = jnp.clip((q - _OFFSET) * (1.0 / _SCALE), 1.0, float(_D - 1))
        e0 = e0.astype(jnp.int32)
        f0 = e0.astype(jnp.float32)
        p0 = f0 * _SCALE + _OFFSET
        pm1 = (f0 - 1.0) * _SCALE + _OFFSET
        ir = jnp.where(pm1 >= q, e0 - 1, jnp.where(p0 >= q, e0, e0 + 1))
        ir = jnp.clip(ir, 1, _D - 1)
        il = ir - 1
        fil = il.astype(jnp.float32)
        pleft = fil * _SCALE + _OFFSET
        pright = (fil + 1.0) * _SCALE + _OFFSET
        dl = jnp.maximum(q - pleft, 0.0)
        dr = jnp.maximum(pright - q, 0.0)
        bz = (dl == 0.0) & (dr == 0.0)
        dl = jnp.where(bz, 1.0, dl)
        dr = jnp.where(bz, 1.0, dr)
        rcp = 1.0 / (dl + dr)
        return il, dr * rcp, dl * rcp

    @pl.loop(0, nch)
    def _chunk(g):
        base = (w + g * _NW) * _C
        cin = [pltpu.async_copy(xs_hbm.at[pl.ds(base, _C)], xsb, sem_in),
               pltpu.async_copy(ys_hbm.at[pl.ds(base, _C)], ysb, sem_in),
               pltpu.async_copy(zs_hbm.at[pl.ds(base, _C)], zsb, sem_in)]
        for cp in cin:
            cp.wait()

        def phase_a(sb):
            bk = sb % 2

            @plsc.parallel_loop(0, _VSB, carry=jnp.int32(0))
            def _pa(j, cb):
                sl = pl.ds(sb * _SB + j * 16, 16)
                xq = xsb[sl]
                yq = ysb[sl]
                zq = zsb[sl]
                ilx, fxl, fxr = bucket(xq)
                ily, fyl, fyr = bucket(yq)
                ilz, fzl, fzr = bucket(zq)
                wbuf[0, sl] = fxl
                wbuf[1, sl] = fyl
                wbuf[2, sl] = fzl
                fbase = ilx * _SX + ily * _D + ilz
                mz = ilz & 7
                mcol[bk, pl.ds(j * 16, 16)] = mz
                bm = mz == 7
                pcb = plsc.all_reduce_population_count(bm)[0]
                for c4 in range(4):
                    m = fbase + _OFF4[c4]
                    midx[bk, pl.ds(c4 * _SB + j * 16, 16)] = m >> 3
                    osl = pl.ds(cb + c4 * pcb, 16)
                    plsc.store_compressed(
                        oidx.at[bk, osl], (m + 1) >> 3, mask=bm)
                    plsc.store_compressed(
                        osid.at[bk, osl],
                        c4 * _C + sb * _SB + j * 16 + lane, mask=bm)
                return cb + 4 * pcb

            return _pa

        def fire(sb, cnt, sem):
            bk = sb % 2
            cps = [pltpu.async_copy(
                grid8_hbm.at[midx.at[bk, pl.ds(r * 128, 128)]],
                mstg.at[bk, pl.ds(r * 128, 128)], sem)
                for r in range(_MFULL)]
            cps.append(pltpu.async_copy(
                grid8_hbm.at[midx.at[bk, pl.ds(_MFULL * 128, _MREM)]],
                mstg.at[bk, pl.ds(_MFULL * 128, _MREM)], sem))
            n_o = (cnt + 127) >> 7

            @pl.loop(0, n_o)
            def _fo(k):
                pltpu.async_copy(
                    grid8_hbm.at[oidx.at[bk, pl.ds(k * 128, 128)]],
                    ostg.at[bk, pl.ds(k * 128, 128)], sem)

            return cps, n_o

        def wait(sb, cps, n_o, sem):
            bk = sb % 2
            for cp in cps:
                cp.wait()

            @pl.loop(0, n_o)
            def _wo(k):
                pltpu.make_async_copy(
                    grid8_hbm.at[pl.ds(0, 128)],
                    ostg.at[bk, pl.ds(0, 128)], sem).wait()

        def scatter_back(sb, cnt):
            bk = sb % 2
            n_t = (cnt + 15) >> 4

            zc = jnp.zeros((16,), jnp.int32)

            @pl.loop(0, n_t)
            def _so(t):
                posn = t * 16 + lane
                mk = posn < cnt
                sidv = osid[bk, pl.ds(t * 16, 16)]
                v = plsc.load_gather(ostg.at[bk], [posn, zc], mask=mk)
                plsc.store_scatter(vrb, [sidv], v, mask=mk)

        def phase_b(sb):
            bk = sb % 2

            @plsc.parallel_loop(0, _VSB)
            def _pb(j):
                sl = pl.ds(sb * _SB + j * 16, 16)
                o = sb * _SB + j * 16
                fxl = wbuf[0, sl]
                fyl = wbuf[1, sl]
                fzl = wbuf[2, sl]
                fxr = 1.0 - fxl
                fyr = 1.0 - fyl
                fzr = 1.0 - fzl
                colv = mcol[bk, pl.ds(j * 16, 16)]
                colv1 = jnp.minimum(colv + 1, 7)
                straddle = colv == 7
                a = []
                for c4 in range(4):
                    pos = c4 * _SB + j * 16 + lane
                    vl = plsc.load_gather(mstg.at[bk], [pos, colv])
                    vr = plsc.load_gather(mstg.at[bk], [pos, colv1])
                    vr = jnp.where(straddle, vrb[pl.ds(c4 * _C + o, 16)], vr)
                    a.append(vl * fzl + vr * fzr)
                b0 = a[0] * fyl + a[1] * fyr
                b1 = a[2] * fyl + a[3] * fyr
                outb[pl.ds(o, 16)] = b0 * fxl + b1 * fxr

        sems = [sem0, sem1]
        inflight = {}
        cnt0 = phase_a(0)
        inflight[0] = (cnt0,) + fire(0, cnt0, sems[0])
        cnt1 = phase_a(1)
        inflight[1] = (cnt1,) + fire(1, cnt1, sems[1])
        for sb in range(2, _NSB):
            pcnt, pcps, pno = inflight.pop(sb - 2)
            wait(sb - 2, pcps, pno, sems[sb % 2])
            scatter_back(sb - 2, pcnt)
            phase_b(sb - 2)
            cnt = phase_a(sb)
            inflight[sb] = (cnt,) + fire(sb, cnt, sems[sb % 2])
        for sb in (_NSB - 2, _NSB - 1):
            pcnt, pcps, pno = inflight.pop(sb)
            wait(sb, pcps, pno, sems[sb % 2])
            scatter_back(sb, pcnt)
            phase_b(sb)

        pltpu.sync_copy(outb, out_hbm.at[pl.ds(base, _C)])


def kernel(x, sdf_grid, x_pts, y_pts, z_pts):
    x = x.reshape(-1, 3).astype(jnp.float32)
    n = x.shape[0]
    xs, ys, zs = x[:, 0], x[:, 1], x[:, 2]
    grid8 = sdf_grid.astype(jnp.float32).reshape(-1, 8)
    mesh = plsc.VectorSubcoreMesh(core_axis_name="c", subcore_axis_name="s")
    run = pl.kernel(
        _body,
        out_type=jax.ShapeDtypeStruct((n,), jnp.float32),
        mesh=mesh,
        compiler_params=pltpu.CompilerParams(
            needs_layout_passes=False, use_tc_tiling_on_sc=False),
        scratch_types=[
            pltpu.VMEM((_C,), jnp.float32),
            pltpu.VMEM((_C,), jnp.float32),
            pltpu.VMEM((_C,), jnp.float32),
            pltpu.VMEM((3, _C), jnp.float32),
            pltpu.VMEM((2, _SB), jnp.int32),
            pltpu.VMEM((2, _ML), jnp.int32),
            pltpu.VMEM((2, _ML, 8), jnp.float32),
            pltpu.VMEM((2, _OCAP), jnp.int32),
            pltpu.VMEM((2, _OCAP), jnp.int32),
            pltpu.VMEM((2, _OCAP, 8), jnp.float32),
            pltpu.VMEM((4 * _C,), jnp.float32),
            pltpu.VMEM((_C,), jnp.float32),
            pltpu.SemaphoreType.DMA,
            pltpu.SemaphoreType.DMA,
            pltpu.SemaphoreType.DMA,
        ],
    )
    return run(xs, ys, zs, grid8)
